# Initial kernel scaffold; baseline (speedup 1.0000x reference)
import functools
import math

import jax
import jax.numpy as jnp
import numpy as np
from jax import lax
from jax.experimental import pallas as pl
from jax.experimental.pallas import tpu as pltpu
from jax.experimental.pallas import tpu_sc as plsc

VOCAB = 1000000
D = 64
BATCH = 16384
SEQ = 50

NC, NS = 2, 16
NW = NC * NS
ROWS = BATCH * SEQ
RPW = ROWS // NW
CH = 100
NCH = RPW // CH
SCALE = math.sqrt(D)


def _positional(seq_len, d_model):
    pe = np.array([
        [pos / np.power(10000, 2 * (k // 2) / d_model) for k in range(d_model)]
        if pos != 0 else np.zeros(d_model)
        for pos in range(seq_len)
    ])
    pe[1:, 0::2] = np.sin(pe[1:, 0::2])
    pe[1:, 1::2] = np.cos(pe[1:, 1::2])
    return jnp.asarray(pe, dtype=jnp.float32)


_POS = _positional(SEQ, D)


@functools.partial(
    pl.kernel,
    out_type=jax.ShapeDtypeStruct((ROWS, D), jnp.float32),
    mesh=plsc.VectorSubcoreMesh(core_axis_name="c", subcore_axis_name="s"),
    scratch_types=[
        pltpu.VMEM((NCH, CH), jnp.int32),
        pltpu.VMEM((SEQ, D), jnp.float32),
        pltpu.VMEM((CH, D), jnp.float32),
        pltpu.SemaphoreType.DMA,
    ],
)
def _emb_lookup(idx_hbm, table_hbm, pos_hbm, out_hbm, idx_v, pos_v, rows_v, gsem):
    wid = lax.axis_index("s") * NC + lax.axis_index("c")
    pltpu.sync_copy(idx_hbm.at[wid], idx_v)
    pltpu.sync_copy(pos_hbm, pos_v)

    def chunk_body(c, carry):
        pltpu.async_copy(table_hbm.at[idx_v.at[c]], rows_v, gsem).wait()

        def row_body(r, rcarry):
            s = lax.rem(r, SEQ)
            for k in range(D // 16):
                sl = pl.ds(k * 16, 16)
                rows_v[r, sl] = rows_v[r, sl] * SCALE + pos_v[s, sl]
            return rcarry

        lax.fori_loop(0, CH, row_body, 0)
        row0 = wid * RPW + c * CH
        pltpu.sync_copy(rows_v, out_hbm.at[pl.ds(row0, CH)])
        return carry

    lax.fori_loop(0, NCH, chunk_body, 0)


def kernel(input_seq, table):
    assert input_seq.shape == (BATCH, SEQ) and table.shape == (VOCAB, D)
    idx = input_seq.reshape(NW, NCH, CH)
    out = _emb_lookup(idx, table, _POS)
    return out.reshape(BATCH, SEQ, D)


# speedup vs baseline: 1.2653x; 1.2653x over previous
import functools
import math

import jax
import jax.numpy as jnp
import numpy as np
from jax import lax
from jax.experimental import pallas as pl
from jax.experimental.pallas import tpu as pltpu
from jax.experimental.pallas import tpu_sc as plsc

VOCAB = 1000000
D = 64
BATCH = 16384
SEQ = 50

NC, NS = 2, 16
NW = NC * NS
ROWS = BATCH * SEQ
RPW = ROWS // NW
CH = 128
NCH = RPW // CH
SCALE = math.sqrt(D)


def _positional(seq_len, d_model):
    pe = np.array([
        [pos / np.power(10000, 2 * (k // 2) / d_model) for k in range(d_model)]
        if pos != 0 else np.zeros(d_model)
        for pos in range(seq_len)
    ])
    pe[1:, 0::2] = np.sin(pe[1:, 0::2])
    pe[1:, 1::2] = np.cos(pe[1:, 1::2])
    return pe.astype(np.float32)


_POS = _positional(SEQ, D)


@functools.partial(
    pl.kernel,
    out_type=jax.ShapeDtypeStruct((ROWS, D), jnp.float32),
    mesh=plsc.VectorSubcoreMesh(core_axis_name="c", subcore_axis_name="s"),
    scratch_types=[
        pltpu.VMEM((NCH, CH), jnp.int32),
        pltpu.VMEM((SEQ, D), jnp.float32),
        pltpu.VMEM((CH, D), jnp.float32),
        pltpu.SemaphoreType.DMA,
    ],
    compiler_params=pltpu.CompilerParams(use_tc_tiling_on_sc=False),
)
def _emb_lookup(idx_hbm, table_hbm, pos_hbm, out_hbm, idx_v, pos_v, rows_v, gsem):
    wid = lax.axis_index("s") * NC + lax.axis_index("c")
    pltpu.sync_copy(idx_hbm.at[wid], idx_v)
    pltpu.sync_copy(pos_hbm, pos_v)

    def chunk_body(c, carry):
        pltpu.async_copy(table_hbm.at[idx_v.at[c]], rows_v, gsem).wait()
        s0 = lax.rem(c * CH, SEQ)

        def row_body(r, rcarry):
            s = lax.rem(s0 + r, SEQ)
            for k in range(D // 16):
                sl = pl.ds(k * 16, 16)
                rows_v[r, sl] = rows_v[r, sl] * SCALE + pos_v[s, sl]
            return rcarry

        lax.fori_loop(0, CH, row_body, 0)
        row0 = wid * RPW + c * CH
        pltpu.sync_copy(rows_v, out_hbm.at[pl.ds(row0, CH)])
        return carry

    lax.fori_loop(0, NCH, chunk_body, 0)


def kernel(input_seq, table):
    assert input_seq.shape == (BATCH, SEQ) and table.shape == (VOCAB, D)
    idx = input_seq.reshape(NW, NCH, CH)
    out = _emb_lookup(idx, table, jnp.asarray(_POS))
    return out.reshape(BATCH, SEQ, D)


# speedup vs baseline: 1.4262x; 1.1272x over previous
import functools
import math

import jax
import jax.numpy as jnp
import numpy as np
from jax import lax
from jax.experimental import pallas as pl
from jax.experimental.pallas import tpu as pltpu
from jax.experimental.pallas import tpu_sc as plsc

VOCAB = 1000000
D = 64
BATCH = 16384
SEQ = 50

NC, NS = 2, 16
NW = NC * NS
ROWS = BATCH * SEQ
RPW = ROWS // NW
CH = 128
NCH = RPW // CH
SCALE = math.sqrt(D)


def _positional(seq_len, d_model):
    pe = np.array([
        [pos / np.power(10000, 2 * (k // 2) / d_model) for k in range(d_model)]
        if pos != 0 else np.zeros(d_model)
        for pos in range(seq_len)
    ])
    pe[1:, 0::2] = np.sin(pe[1:, 0::2])
    pe[1:, 1::2] = np.cos(pe[1:, 1::2])
    return pe.astype(np.float32)


_POS = _positional(SEQ, D)


@functools.partial(
    pl.kernel,
    out_type=jax.ShapeDtypeStruct((ROWS, D), jnp.float32),
    mesh=plsc.VectorSubcoreMesh(core_axis_name="c", subcore_axis_name="s"),
    scratch_types=[
        pltpu.VMEM((NCH, CH), jnp.int32),
        pltpu.VMEM((SEQ, D), jnp.float32),
        pltpu.VMEM((CH, D), jnp.float32),
        pltpu.VMEM((CH, D), jnp.float32),
        pltpu.SemaphoreType.DMA,
        pltpu.SemaphoreType.DMA,
        pltpu.SemaphoreType.DMA,
        pltpu.SemaphoreType.DMA,
    ],
    compiler_params=pltpu.CompilerParams(use_tc_tiling_on_sc=False),
)
def _emb_lookup(idx_hbm, table_hbm, pos_hbm, out_hbm,
                idx_v, pos_v, rb0, rb1, g0, g1, o0, o1):
    wid = lax.axis_index("s") * NC + lax.axis_index("c")
    pltpu.sync_copy(idx_hbm.at[wid], idx_v)
    pltpu.sync_copy(pos_hbm, pos_v)

    pltpu.async_copy(table_hbm.at[idx_v.at[0]], rb0, g0)
    pltpu.async_copy(table_hbm.at[idx_v.at[1]], rb1, g1)

    def process(cc, rb, gs, os):
        pltpu.make_async_copy(table_hbm.at[idx_v.at[cc]], rb, gs).wait()
        s0 = lax.rem(cc * CH, SEQ)

        def row_body(r, rcarry):
            s = lax.rem(s0 + r, SEQ)
            for k in range(D // 16):
                sl = pl.ds(k * 16, 16)
                rb[r, sl] = rb[r, sl] * SCALE + pos_v[s, sl]
            return rcarry

        lax.fori_loop(0, CH, row_body, 0)
        row0 = wid * RPW + cc * CH
        pltpu.async_copy(rb, out_hbm.at[pl.ds(row0, CH)], os).wait()
        nxt = cc + 2

        @pl.when(nxt < NCH)
        def _():
            pltpu.async_copy(table_hbm.at[idx_v.at[nxt]], rb, gs)

    def pair_body(i, carry):
        process(2 * i, rb0, g0, o0)
        process(2 * i + 1, rb1, g1, o1)
        return carry

    lax.fori_loop(0, NCH // 2, pair_body, 0)


def kernel(input_seq, table):
    assert input_seq.shape == (BATCH, SEQ) and table.shape == (VOCAB, D)
    idx = input_seq.reshape(NW, NCH, CH)
    out = _emb_lookup(idx, table, jnp.asarray(_POS))
    return out.reshape(BATCH, SEQ, D)


# speedup vs baseline: 1.4821x; 1.0392x over previous
import functools
import math

import jax
import jax.numpy as jnp
import numpy as np
from jax import lax
from jax.experimental import pallas as pl
from jax.experimental.pallas import tpu as pltpu
from jax.experimental.pallas import tpu_sc as plsc

VOCAB = 1000000
D = 64
BATCH = 16384
SEQ = 50

NC, NS = 2, 16
NW = NC * NS
ROWS = BATCH * SEQ
RPW = ROWS // NW
CH = 128
NCH = RPW // CH
SCALE = math.sqrt(D)


def _positional(seq_len, d_model):
    pe = np.array([
        [pos / np.power(10000, 2 * (k // 2) / d_model) for k in range(d_model)]
        if pos != 0 else np.zeros(d_model)
        for pos in range(seq_len)
    ])
    pe[1:, 0::2] = np.sin(pe[1:, 0::2])
    pe[1:, 1::2] = np.cos(pe[1:, 1::2])
    return pe.astype(np.float32)


_POS = _positional(SEQ, D)


@functools.partial(
    pl.kernel,
    out_type=jax.ShapeDtypeStruct((ROWS, D), jnp.float32),
    mesh=plsc.VectorSubcoreMesh(core_axis_name="c", subcore_axis_name="s"),
    scratch_types=[
        pltpu.VMEM((NCH, CH), jnp.int32),
        pltpu.VMEM((4 * SEQ, D), jnp.float32),
        pltpu.VMEM((CH, D), jnp.float32),
        pltpu.VMEM((CH, D), jnp.float32),
        pltpu.VMEM((CH, D), jnp.float32),
        pltpu.VMEM((CH, D), jnp.float32),
        pltpu.SemaphoreType.DMA,
        pltpu.SemaphoreType.DMA,
        pltpu.SemaphoreType.DMA,
        pltpu.SemaphoreType.DMA,
        pltpu.SemaphoreType.DMA,
        pltpu.SemaphoreType.DMA,
        pltpu.SemaphoreType.DMA,
        pltpu.SemaphoreType.DMA,
    ],
    compiler_params=pltpu.CompilerParams(use_tc_tiling_on_sc=False),
)
def _emb_lookup(idx_hbm, table_hbm, pos_hbm, out_hbm,
                idx_v, pos_v, rb0, rb1, rb2, rb3,
                g0, g1, g2, g3, o0, o1, o2, o3):
    wid = lax.axis_index("s") * NC + lax.axis_index("c")
    rbs = (rb0, rb1, rb2, rb3)
    gs = (g0, g1, g2, g3)
    os_ = (o0, o1, o2, o3)
    pltpu.sync_copy(idx_hbm.at[wid], idx_v)
    for j in range(4):
        pltpu.sync_copy(pos_hbm, pos_v.at[pl.ds(j * SEQ, SEQ)])

    for c in range(3):
        pltpu.async_copy(table_hbm.at[idx_v.at[c]], rbs[c], gs[c])

    def process(c, b):
        rb = rbs[b]
        pltpu.make_async_copy(table_hbm.at[idx_v.at[c]], rb, gs[b]).wait()
        s0 = lax.rem(c * CH, SEQ)

        def row_body(r, rcarry):
            for k in range(D // 16):
                sl = pl.ds(k * 16, 16)
                rb[r, sl] = rb[r, sl] * SCALE + pos_v[s0 + r, sl]
            return rcarry

        lax.fori_loop(0, CH, row_body, 0, unroll=4)
        row0 = wid * RPW + c * CH
        pltpu.async_copy(rb, out_hbm.at[pl.ds(row0, CH)], os_[b])

        nxt = c + 3
        bn = (b + 3) % 4

        @pl.when(jnp.logical_and(nxt < NCH, c >= 1))
        def _():
            pltpu.make_async_copy(
                rbs[bn], out_hbm.at[pl.ds(wid * RPW + (c - 1) * CH, CH)],
                os_[bn]).wait()

        @pl.when(nxt < NCH)
        def _():
            pltpu.async_copy(table_hbm.at[idx_v.at[nxt]], rbs[bn], gs[bn])

    def quad_body(i, carry):
        for b in range(4):
            process(4 * i + b, b)
        return carry

    lax.fori_loop(0, NCH // 4, quad_body, 0)

    for b in range(4):
        c = NCH - 4 + b
        pltpu.make_async_copy(
            rbs[b], out_hbm.at[pl.ds(wid * RPW + c * CH, CH)], os_[b]).wait()


def kernel(input_seq, table):
    assert input_seq.shape == (BATCH, SEQ) and table.shape == (VOCAB, D)
    idx = input_seq.reshape(NW, NCH, CH)
    out = _emb_lookup(idx, table, jnp.asarray(_POS))
    return out.reshape(BATCH, SEQ, D)
